# baseline (speedup 1.0000x reference)
import functools

import jax
import jax.numpy as jnp
from jax import lax
from jax.experimental import pallas as pl
from jax.experimental.pallas import tpu as pltpu
from jax.experimental.pallas import tpu_sc as plsc

_N = 10000
_E = 320000
_H = 64

_NC = 2
_NS = 16
_NW = _NC * _NS
_NPAD = 10240
_RPT = _NPAD // _NS
_ZC = 32
_W = 128
_EB = 80
_EW = _E // _NW
_NCH = _EW // _EB

def _deg_body(dst_hbm, out_hbm, acc_sh, dst_v, ones_v, zero_v):
    cid = lax.axis_index("c")
    sid = lax.axis_index("s")
    for j in range(_EB // 16):
        ones_v[pl.ds(j * 16, 16)] = jnp.ones((16,), jnp.float32)
    for j in range(_ZC // 16):
        zero_v[pl.ds(j * 16, 16)] = jnp.zeros((16,), jnp.float32)
    rbase = sid * _RPT

    def zloop(k, car):
        pltpu.sync_copy(zero_v, acc_sh.at[pl.ds(rbase + k * _ZC, _ZC)])
        return car

    lax.fori_loop(0, _RPT // _ZC, zloop, 0)
    plsc.subcore_barrier()

    ebase = (cid * _NS + sid) * _EW

    def eloop(k, car):
        pltpu.sync_copy(dst_hbm.at[pl.ds(ebase + k * _EB, _EB)], dst_v)
        pltpu.sync_copy(ones_v, acc_sh.at[dst_v], add=True)
        return car

    lax.fori_loop(0, _NCH, eloop, 0)
    plsc.subcore_barrier()
    pltpu.sync_copy(acc_sh.at[pl.ds(rbase, _RPT)],
                    out_hbm.at[cid, pl.ds(rbase, _RPT)])


@functools.cache
def _get_deg_kernel():
    mesh = plsc.VectorSubcoreMesh(core_axis_name="c", subcore_axis_name="s")
    return pl.kernel(
        _deg_body,
        out_type=jax.ShapeDtypeStruct((_NC, _NPAD), jnp.float32),
        mesh=mesh,
        scratch_types=[
            pltpu.VMEM_SHARED((_NPAD,), jnp.float32),
            pltpu.VMEM((_EB,), jnp.int32),
            pltpu.VMEM((_EB,), jnp.float32),
            pltpu.VMEM((_ZC,), jnp.float32),
        ],
    )


_CPS = 8
_NSL = 10
_EWP = _NSL * _CPS * 128
_EP = _NW * _EWP


def _conv_body(g_hbm, src_hbm, dst_hbm, out_hbm,
               acc_sh, src_i, dst_i, rows, isem, gsem):
    cid = lax.axis_index("c")
    sid = lax.axis_index("s")
    w = cid * _NS + sid
    rbase = sid * _RPT

    def zfill(r, car):
        for j in range(_W // 16):
            rows[0, r, pl.ds(j * 16, 16)] = jnp.zeros((16,), jnp.float32)
        return car
    lax.fori_loop(0, 128, zfill, 0)

    def zloop(k, car):
        pltpu.sync_copy(rows.at[0], acc_sh.at[pl.ds(rbase + k * 128, 128)])
        return car
    lax.fori_loop(0, _RPT // 128, zloop, 0)
    plsc.subcore_barrier()

    def sloop(s, car):
        sb = lax.rem(s, 2)
        pltpu.sync_copy(src_hbm.at[w, s], src_i.at[sb])
        pltpu.sync_copy(dst_hbm.at[w, s], dst_i.at[sb])
        for j in range(_CPS):
            b = j % 2
            pltpu.async_copy(g_hbm.at[src_i.at[sb, j]], rows.at[b], gsem).wait()
            pltpu.sync_copy(rows.at[b], acc_sh.at[dst_i.at[sb, j]], add=True)
        return car

    lax.fori_loop(0, _NSL, sloop, 0)
    plsc.subcore_barrier()
    pltpu.sync_copy(acc_sh.at[pl.ds(rbase, _RPT)],
                    out_hbm.at[cid, pl.ds(rbase, _RPT)])


@functools.cache
def _get_conv_kernel():
    mesh = plsc.VectorSubcoreMesh(core_axis_name="c", subcore_axis_name="s")
    return pl.kernel(
        _conv_body,
        out_type=jax.ShapeDtypeStruct((_NC, _NPAD, _W), jnp.float32),
        mesh=mesh,
        scratch_types=[
            pltpu.VMEM_SHARED((_NPAD, _W), jnp.float32),
            pltpu.VMEM((2, _CPS, 128), jnp.int32),
            pltpu.VMEM((2, _CPS, 128), jnp.int32),
            pltpu.VMEM((2, 128, _W), jnp.float32),
            pltpu.SemaphoreType.DMA,
            pltpu.SemaphoreType.DMA,
        ],
    )

_RB = 1000
_GRID = _N // _RB


def _dinv_from(degp_ref):
    deg = degp_ref[0] + degp_ref[1]
    return jnp.where(deg > 0, lax.rsqrt(jnp.maximum(deg, 1.0)), 0.0)


def _stage1_body(x_ref, w1_ref, b1_ref, degp_ref, h_ref, g_ref):
    h = jnp.maximum(
        jnp.dot(x_ref[...], w1_ref[...], preferred_element_type=jnp.float32)
        + b1_ref[...], 0.0)
    dinv = _dinv_from(degp_ref)
    h_ref[...] = h
    g_ref[...] = jnp.pad(dinv * h, ((0, 0), (0, _W - _H)))


_stage1 = pl.pallas_call(
    _stage1_body,
    grid=(_GRID,),
    in_specs=[
        pl.BlockSpec((_RB, 128), lambda i: (i, 0)),
        pl.BlockSpec((128, _H), lambda i: (0, 0)),
        pl.BlockSpec((1, _H), lambda i: (0, 0)),
        pl.BlockSpec((_NC, _RB, 1), lambda i: (0, i, 0)),
    ],
    out_specs=[
        pl.BlockSpec((_RB, _H), lambda i: (i, 0)),
        pl.BlockSpec((_RB, _W), lambda i: (i, 0)),
    ],
    out_shape=[
        jax.ShapeDtypeStruct((_N, _H), jnp.float32),
        jax.ShapeDtypeStruct((_N, _W), jnp.float32),
    ],
)


def _mid_body(h_ref, parts_ref, degp_ref, wt_ref, wb_ref, aa_ref, ab_ref,
              hn_ref, gn_ref):
    h = h_ref[...]
    dinv = _dinv_from(degp_ref)
    c = dinv * (parts_ref[0, :, :_H] + parts_ref[1, :, :_H])
    z = (jnp.dot(h, wt_ref[...], preferred_element_type=jnp.float32)
         + jnp.dot(c, wb_ref[...], preferred_element_type=jnp.float32)
         + jnp.dot(h, aa_ref[...], preferred_element_type=jnp.float32)
         * jnp.dot(h, ab_ref[...], preferred_element_type=jnp.float32))
    hn = jnp.maximum(z, 0.0)
    hn_ref[...] = hn
    gn_ref[...] = jnp.pad(dinv * hn, ((0, 0), (0, _W - _H)))


def _last_body(h_ref, parts_ref, degp_ref, wt_ref, wb_ref, aa_ref, ab_ref,
               out_ref):
    h = h_ref[...]
    dinv = _dinv_from(degp_ref)
    c = dinv * (parts_ref[0, :, :_H] + parts_ref[1, :, :_H])
    z = (jnp.dot(h, wt_ref[...], preferred_element_type=jnp.float32)
         + jnp.dot(c, wb_ref[...], preferred_element_type=jnp.float32)
         + jnp.dot(h, aa_ref[...], preferred_element_type=jnp.float32)
         * jnp.dot(h, ab_ref[...], preferred_element_type=jnp.float32))
    out_ref[...] = jnp.maximum(z, 0.0)


_mid_in_specs = [
    pl.BlockSpec((_RB, _H), lambda i: (i, 0)),
    pl.BlockSpec((_NC, _RB, _W), lambda i: (0, i, 0)),
    pl.BlockSpec((_NC, _RB, 1), lambda i: (0, i, 0)),
    pl.BlockSpec((_H, _H), lambda i: (0, 0)),
    pl.BlockSpec((_H, _H), lambda i: (0, 0)),
    pl.BlockSpec((_H, _H), lambda i: (0, 0)),
    pl.BlockSpec((_H, _H), lambda i: (0, 0)),
]

_stage_mid = pl.pallas_call(
    _mid_body,
    grid=(_GRID,),
    in_specs=_mid_in_specs,
    out_specs=[
        pl.BlockSpec((_RB, _H), lambda i: (i, 0)),
        pl.BlockSpec((_RB, _W), lambda i: (i, 0)),
    ],
    out_shape=[
        jax.ShapeDtypeStruct((_N, _H), jnp.float32),
        jax.ShapeDtypeStruct((_N, _W), jnp.float32),
    ],
)

_stage_last = pl.pallas_call(
    _last_body,
    grid=(_GRID,),
    in_specs=_mid_in_specs,
    out_specs=pl.BlockSpec((_RB, _H), lambda i: (i, 0)),
    out_shape=jax.ShapeDtypeStruct((_N, _H), jnp.float32),
)


def kernel(x, edge_index, W1, b1, W2, A2a, A2b, W3, A3a, A3b, W4, A4a, A4b):
    src = edge_index[0].astype(jnp.int32)
    dst = edge_index[1].astype(jnp.int32)
    srcp = jnp.concatenate([src, jnp.zeros(_EP - _E, jnp.int32)])
    dstp = jnp.concatenate([dst, jnp.full(_EP - _E, _N, jnp.int32)])
    srcp = srcp.reshape(_NW, _NSL, _CPS, 128)
    dstp = dstp.reshape(_NW, _NSL, _CPS, 128)
    deg_kernel = _get_deg_kernel()
    conv_kernel = _get_conv_kernel()
    deg_parts = deg_kernel(dst).reshape(_NC, _NPAD, 1)
    h1, g1 = _stage1(x, W1, b1.reshape(1, _H), deg_parts)
    p1 = conv_kernel(g1, srcp, dstp)
    h2, g2 = _stage_mid(h1, p1, deg_parts, W2[:_H], W2[_H:], A2a, A2b)
    p2 = conv_kernel(g2, srcp, dstp)
    h3, g3 = _stage_mid(h2, p2, deg_parts, W3[:_H], W3[_H:], A3a, A3b)
    p3 = conv_kernel(g3, srcp, dstp)
    out = _stage_last(h3, p3, deg_parts, W4[:_H], W4[_H:], A4a, A4b)
    return out


# speedup vs baseline: 2.7485x; 2.7485x over previous
import functools

import jax
import jax.numpy as jnp
from jax import lax
from jax.experimental import pallas as pl
from jax.experimental.pallas import tpu as pltpu
from jax.experimental.pallas import tpu_sc as plsc

_N = 10000
_E = 320000
_H = 64

_NC = 2
_NS = 16
_NW = _NC * _NS
_NPAD = 10240
_RPT = _NPAD // _NS
_ZC = 32
_W = 128
_EB = 80
_EW = _E // _NW
_NCH = _EW // _EB

def _deg_body(dst_hbm, out_hbm, acc_sh, dst_v, ones_v, zero_v):
    cid = lax.axis_index("c")
    sid = lax.axis_index("s")
    for j in range(_EB // 16):
        ones_v[pl.ds(j * 16, 16)] = jnp.ones((16,), jnp.float32)
    for j in range(_ZC // 16):
        zero_v[pl.ds(j * 16, 16)] = jnp.zeros((16,), jnp.float32)
    rbase = sid * _RPT

    def zloop(k, car):
        pltpu.sync_copy(zero_v, acc_sh.at[pl.ds(rbase + k * _ZC, _ZC)])
        return car

    lax.fori_loop(0, _RPT // _ZC, zloop, 0)
    plsc.subcore_barrier()

    ebase = (cid * _NS + sid) * _EW

    def eloop(k, car):
        pltpu.sync_copy(dst_hbm.at[pl.ds(ebase + k * _EB, _EB)], dst_v)
        pltpu.sync_copy(ones_v, acc_sh.at[dst_v], add=True)
        return car

    lax.fori_loop(0, _NCH, eloop, 0)
    plsc.subcore_barrier()
    pltpu.sync_copy(acc_sh.at[pl.ds(rbase, _RPT)],
                    out_hbm.at[cid, pl.ds(rbase, _RPT)])


@functools.cache
def _get_deg_kernel():
    mesh = plsc.VectorSubcoreMesh(core_axis_name="c", subcore_axis_name="s")
    return pl.kernel(
        _deg_body,
        out_type=jax.ShapeDtypeStruct((_NC, _NPAD), jnp.float32),
        mesh=mesh,
        scratch_types=[
            pltpu.VMEM_SHARED((_NPAD,), jnp.float32),
            pltpu.VMEM((_EB,), jnp.int32),
            pltpu.VMEM((_EB,), jnp.float32),
            pltpu.VMEM((_ZC,), jnp.float32),
        ],
    )


def _conv_body(g_hbm, src_hbm, dst_hbm, out_hbm,
               acc_sh, src0, src1, dst0, dst1, rows0, rows1, zero_v,
               isem, gsem):
    cid = lax.axis_index("c")
    sid = lax.axis_index("s")
    w = cid * _NS + sid
    rbase = sid * _RPT

    for r in range(_ZC):
        for j in range(_W // 16):
            zero_v[r, pl.ds(j * 16, 16)] = jnp.zeros((16,), jnp.float32)

    def zloop(k, car):
        pltpu.sync_copy(zero_v, acc_sh.at[pl.ds(rbase + k * _ZC, _ZC)])
        return car
    lax.fori_loop(0, _RPT // _ZC, zloop, 0)
    plsc.subcore_barrier()

    ebase = w * _EW

    def idx_start(c, sbuf, dbuf):
        pltpu.async_copy(src_hbm.at[pl.ds(ebase + c * _EB, _EB)], sbuf, isem)
        pltpu.async_copy(dst_hbm.at[pl.ds(ebase + c * _EB, _EB)], dbuf, isem)

    def idx_wait(sbuf, dbuf):
        pltpu.make_async_copy(src_hbm.at[pl.ds(ebase, _EB)], sbuf, isem).wait()
        pltpu.make_async_copy(dst_hbm.at[pl.ds(ebase, _EB)], dbuf, isem).wait()

    def gather_start(sbuf, rbuf):
        pltpu.async_copy(g_hbm.at[sbuf], rbuf, gsem)

    def gather_wait(sbuf, rbuf):
        pltpu.make_async_copy(g_hbm.at[sbuf], rbuf, gsem).wait()

    pltpu.sync_copy(src_hbm.at[pl.ds(ebase, _EB)], src0)
    pltpu.sync_copy(dst_hbm.at[pl.ds(ebase, _EB)], dst0)
    gather_start(src0, rows0)
    idx_start(1, src1, dst1)

    _HALF = (_NCH - 1) // 2

    def ploop(i, car):
        gather_wait(src0, rows0)
        idx_wait(src1, dst1)
        gather_start(src1, rows1)
        pltpu.sync_copy(rows0, acc_sh.at[dst0], add=True)
        idx_start(2 * i + 2, src0, dst0)
        gather_wait(src1, rows1)
        idx_wait(src0, dst0)
        gather_start(src0, rows0)
        pltpu.sync_copy(rows1, acc_sh.at[dst1], add=True)

        @pl.when(i < _HALF - 1)
        def _():
            idx_start(2 * i + 3, src1, dst1)
        return car

    lax.fori_loop(0, _HALF, ploop, 0)
    gather_wait(src0, rows0)
    pltpu.sync_copy(rows0, acc_sh.at[dst0], add=True)

    plsc.subcore_barrier()
    pltpu.sync_copy(acc_sh.at[pl.ds(rbase, _RPT)],
                    out_hbm.at[cid, pl.ds(rbase, _RPT)])


@functools.cache
def _get_conv_kernel():
    mesh = plsc.VectorSubcoreMesh(core_axis_name="c", subcore_axis_name="s")
    return pl.kernel(
        _conv_body,
        out_type=jax.ShapeDtypeStruct((_NC, _NPAD, _W), jnp.float32),
        mesh=mesh,
        scratch_types=[
            pltpu.VMEM_SHARED((_NPAD, _W), jnp.float32),
            pltpu.VMEM((_EB,), jnp.int32),
            pltpu.VMEM((_EB,), jnp.int32),
            pltpu.VMEM((_EB,), jnp.int32),
            pltpu.VMEM((_EB,), jnp.int32),
            pltpu.VMEM((_EB, _W), jnp.float32),
            pltpu.VMEM((_EB, _W), jnp.float32),
            pltpu.VMEM((_ZC, _W), jnp.float32),
            pltpu.SemaphoreType.DMA,
            pltpu.SemaphoreType.DMA,
        ],
    )

_RB = 1000
_GRID = _N // _RB


def _dinv_from(degp_ref):
    deg = degp_ref[0] + degp_ref[1]
    return jnp.where(deg > 0, lax.rsqrt(jnp.maximum(deg, 1.0)), 0.0)


def _stage1_body(x_ref, w1_ref, b1_ref, degp_ref, h_ref, g_ref):
    h = jnp.maximum(
        jnp.dot(x_ref[...], w1_ref[...], preferred_element_type=jnp.float32)
        + b1_ref[...], 0.0)
    dinv = _dinv_from(degp_ref)
    h_ref[...] = h
    g_ref[...] = jnp.pad(dinv * h, ((0, 0), (0, _W - _H)))


_stage1 = pl.pallas_call(
    _stage1_body,
    grid=(_GRID,),
    in_specs=[
        pl.BlockSpec((_RB, 128), lambda i: (i, 0)),
        pl.BlockSpec((128, _H), lambda i: (0, 0)),
        pl.BlockSpec((1, _H), lambda i: (0, 0)),
        pl.BlockSpec((_NC, _RB, 1), lambda i: (0, i, 0)),
    ],
    out_specs=[
        pl.BlockSpec((_RB, _H), lambda i: (i, 0)),
        pl.BlockSpec((_RB, _W), lambda i: (i, 0)),
    ],
    out_shape=[
        jax.ShapeDtypeStruct((_N, _H), jnp.float32),
        jax.ShapeDtypeStruct((_N, _W), jnp.float32),
    ],
)


def _mid_body(h_ref, parts_ref, degp_ref, wt_ref, wb_ref, aa_ref, ab_ref,
              hn_ref, gn_ref):
    h = h_ref[...]
    dinv = _dinv_from(degp_ref)
    c = dinv * (parts_ref[0, :, :_H] + parts_ref[1, :, :_H])
    z = (jnp.dot(h, wt_ref[...], preferred_element_type=jnp.float32)
         + jnp.dot(c, wb_ref[...], preferred_element_type=jnp.float32)
         + jnp.dot(h, aa_ref[...], preferred_element_type=jnp.float32)
         * jnp.dot(h, ab_ref[...], preferred_element_type=jnp.float32))
    hn = jnp.maximum(z, 0.0)
    hn_ref[...] = hn
    gn_ref[...] = jnp.pad(dinv * hn, ((0, 0), (0, _W - _H)))


def _last_body(h_ref, parts_ref, degp_ref, wt_ref, wb_ref, aa_ref, ab_ref,
               out_ref):
    h = h_ref[...]
    dinv = _dinv_from(degp_ref)
    c = dinv * (parts_ref[0, :, :_H] + parts_ref[1, :, :_H])
    z = (jnp.dot(h, wt_ref[...], preferred_element_type=jnp.float32)
         + jnp.dot(c, wb_ref[...], preferred_element_type=jnp.float32)
         + jnp.dot(h, aa_ref[...], preferred_element_type=jnp.float32)
         * jnp.dot(h, ab_ref[...], preferred_element_type=jnp.float32))
    out_ref[...] = jnp.maximum(z, 0.0)


_mid_in_specs = [
    pl.BlockSpec((_RB, _H), lambda i: (i, 0)),
    pl.BlockSpec((_NC, _RB, _W), lambda i: (0, i, 0)),
    pl.BlockSpec((_NC, _RB, 1), lambda i: (0, i, 0)),
    pl.BlockSpec((_H, _H), lambda i: (0, 0)),
    pl.BlockSpec((_H, _H), lambda i: (0, 0)),
    pl.BlockSpec((_H, _H), lambda i: (0, 0)),
    pl.BlockSpec((_H, _H), lambda i: (0, 0)),
]

_stage_mid = pl.pallas_call(
    _mid_body,
    grid=(_GRID,),
    in_specs=_mid_in_specs,
    out_specs=[
        pl.BlockSpec((_RB, _H), lambda i: (i, 0)),
        pl.BlockSpec((_RB, _W), lambda i: (i, 0)),
    ],
    out_shape=[
        jax.ShapeDtypeStruct((_N, _H), jnp.float32),
        jax.ShapeDtypeStruct((_N, _W), jnp.float32),
    ],
)

_stage_last = pl.pallas_call(
    _last_body,
    grid=(_GRID,),
    in_specs=_mid_in_specs,
    out_specs=pl.BlockSpec((_RB, _H), lambda i: (i, 0)),
    out_shape=jax.ShapeDtypeStruct((_N, _H), jnp.float32),
)


def kernel(x, edge_index, W1, b1, W2, A2a, A2b, W3, A3a, A3b, W4, A4a, A4b):
    src = edge_index[0].astype(jnp.int32)
    dst = edge_index[1].astype(jnp.int32)
    deg_kernel = _get_deg_kernel()
    conv_kernel = _get_conv_kernel()
    deg_parts = deg_kernel(dst).reshape(_NC, _NPAD, 1)
    h1, g1 = _stage1(x, W1, b1.reshape(1, _H), deg_parts)
    p1 = conv_kernel(g1, src, dst)
    h2, g2 = _stage_mid(h1, p1, deg_parts, W2[:_H], W2[_H:], A2a, A2b)
    p2 = conv_kernel(g2, src, dst)
    h3, g3 = _stage_mid(h2, p2, deg_parts, W3[:_H], W3[_H:], A3a, A3b)
    p3 = conv_kernel(g3, src, dst)
    out = _stage_last(h3, p3, deg_parts, W4[:_H], W4[_H:], A4a, A4b)
    return out


# speedup vs baseline: 3.6507x; 1.3282x over previous
import functools

import jax
import jax.numpy as jnp
from jax import lax
from jax.experimental import pallas as pl
from jax.experimental.pallas import tpu as pltpu
from jax.experimental.pallas import tpu_sc as plsc

_N = 10000
_E = 320000
_H = 64

_NC = 2
_NS = 16
_NW = _NC * _NS
_NPAD = 10240
_RPT = _NPAD // _NS
_ZC = 32
_W = 128
_EB = 80
_EW = _E // _NW
_NCH = _EW // _EB

def _deg_body(dst_hbm, out_hbm, acc_sh, dst_v, ones_v, zero_v):
    cid = lax.axis_index("c")
    sid = lax.axis_index("s")
    for j in range(_EB // 16):
        ones_v[pl.ds(j * 16, 16)] = jnp.ones((16,), jnp.float32)
    for j in range(_ZC // 16):
        zero_v[pl.ds(j * 16, 16)] = jnp.zeros((16,), jnp.float32)
    rbase = sid * _RPT

    def zloop(k, car):
        pltpu.sync_copy(zero_v, acc_sh.at[pl.ds(rbase + k * _ZC, _ZC)])
        return car

    lax.fori_loop(0, _RPT // _ZC, zloop, 0)
    plsc.subcore_barrier()

    ebase = (cid * _NS + sid) * _EW

    def eloop(k, car):
        pltpu.sync_copy(dst_hbm.at[pl.ds(ebase + k * _EB, _EB)], dst_v)
        pltpu.sync_copy(ones_v, acc_sh.at[dst_v], add=True)
        return car

    lax.fori_loop(0, _NCH, eloop, 0)
    plsc.subcore_barrier()
    pltpu.sync_copy(acc_sh.at[pl.ds(rbase, _RPT)],
                    out_hbm.at[cid, pl.ds(rbase, _RPT)])


@functools.cache
def _get_deg_kernel():
    mesh = plsc.VectorSubcoreMesh(core_axis_name="c", subcore_axis_name="s")
    return pl.kernel(
        _deg_body,
        out_type=jax.ShapeDtypeStruct((_NC, _NPAD), jnp.float32),
        mesh=mesh,
        scratch_types=[
            pltpu.VMEM_SHARED((_NPAD,), jnp.float32),
            pltpu.VMEM((_EB,), jnp.int32),
            pltpu.VMEM((_EB,), jnp.float32),
            pltpu.VMEM((_ZC,), jnp.float32),
        ],
    )


def _conv_body(g_hbm, src_hbm, dst_hbm, out_hbm, acc_sh, src_i, dst_i, rows,
               isem, gsem, ssem):
    cid = lax.axis_index("c")
    sid = lax.axis_index("s")
    w = cid * _NS + sid
    rbase = sid * _RPT
    ebase = w * _EW

    def idx_start(cexpr, s8):
        off = ebase + cexpr * _EB
        pltpu.async_copy(src_hbm.at[pl.ds(off, _EB)], src_i.at[s8], isem)
        pltpu.async_copy(dst_hbm.at[pl.ds(off, _EB)], dst_i.at[s8], isem)

    def idx_wait():
        pltpu.make_async_copy(src_hbm.at[pl.ds(ebase, _EB)], src_i.at[0],
                              isem).wait()
        pltpu.make_async_copy(dst_hbm.at[pl.ds(ebase, _EB)], dst_i.at[0],
                              isem).wait()

    def gather_start(s8, b4):
        pltpu.async_copy(g_hbm.at[src_i.at[s8]], rows.at[b4], gsem)

    def gather_wait():
        pltpu.make_async_copy(g_hbm.at[pl.ds(0, _EB)], rows.at[0], gsem).wait()

    def scatter_start(b4, s8):
        pltpu.async_copy(rows.at[b4], acc_sh.at[dst_i.at[s8]], ssem, add=True)

    def scatter_wait():
        pltpu.make_async_copy(rows.at[0], acc_sh.at[pl.ds(0, _EB)], ssem).wait()

    def chunk_ops(cexpr, cm, s3=True, s45=True, s6=True):
        gather_wait()
        scatter_start(cm % 4, cm % 8)
        if s3:
            scatter_wait()
        if s45:
            idx_wait()
            gather_start((cm + 2) % 8, (cm + 2) % 4)
        if s6:
            idx_start(cexpr + 6, (cm + 6) % 8)

    def zfill(r, car):
        for j in range(_W // 16):
            rows[0, r, pl.ds(j * 16, 16)] = jnp.zeros((16,), jnp.float32)
        return car
    lax.fori_loop(0, _EB, zfill, 0)

    def zloop(k, car):
        pltpu.sync_copy(rows.at[0], acc_sh.at[pl.ds(rbase + k * _EB, _EB)])
        return car
    lax.fori_loop(0, _RPT // _EB, zloop, 0)
    plsc.subcore_barrier()

    pltpu.sync_copy(src_hbm.at[pl.ds(ebase, _EB)], src_i.at[0])
    pltpu.sync_copy(dst_hbm.at[pl.ds(ebase, _EB)], dst_i.at[0])
    pltpu.sync_copy(src_hbm.at[pl.ds(ebase + _EB, _EB)], src_i.at[1])
    pltpu.sync_copy(dst_hbm.at[pl.ds(ebase + _EB, _EB)], dst_i.at[1])
    for c in (2, 3, 4, 5):
        idx_start(c, c)
    gather_start(0, 0)
    gather_start(1, 1)
    chunk_ops(0, 0, s3=False)
    chunk_ops(1, 1, s3=False)

    def sloop(i, car):
        cbase = 2 + 8 * i
        for j in range(8):
            chunk_ops(cbase + j, 2 + j)
        return car
    lax.fori_loop(0, 14, sloop, 0)

    for c in range(114, 119):
        chunk_ops(c, c)
    for c in range(119, 123):
        chunk_ops(c, c, s6=False)
    for c in (123, 124):
        chunk_ops(c, c, s45=False, s6=False)
    scatter_wait()
    scatter_wait()

    plsc.subcore_barrier()
    pltpu.sync_copy(acc_sh.at[pl.ds(rbase, _RPT)],
                    out_hbm.at[cid, pl.ds(rbase, _RPT)])


@functools.cache
def _get_conv_kernel():
    mesh = plsc.VectorSubcoreMesh(core_axis_name="c", subcore_axis_name="s")
    return pl.kernel(
        _conv_body,
        out_type=jax.ShapeDtypeStruct((_NC, _NPAD, _W), jnp.float32),
        mesh=mesh,
        scratch_types=[
            pltpu.VMEM_SHARED((_NPAD, _W), jnp.float32),
            pltpu.VMEM((8, _EB), jnp.int32),
            pltpu.VMEM((8, _EB), jnp.int32),
            pltpu.VMEM((4, _EB, _W), jnp.float32),
            pltpu.SemaphoreType.DMA,
            pltpu.SemaphoreType.DMA,
            pltpu.SemaphoreType.DMA,
        ],
    )

_RB = 1000
_GRID = _N // _RB


def _dinv_from(degp_ref):
    deg = degp_ref[0] + degp_ref[1]
    return jnp.where(deg > 0, lax.rsqrt(jnp.maximum(deg, 1.0)), 0.0)


def _stage1_body(x_ref, w1_ref, b1_ref, degp_ref, h_ref, g_ref):
    h = jnp.maximum(
        jnp.dot(x_ref[...], w1_ref[...], preferred_element_type=jnp.float32)
        + b1_ref[...], 0.0)
    dinv = _dinv_from(degp_ref)
    h_ref[...] = h
    g_ref[...] = jnp.pad(dinv * h, ((0, 0), (0, _W - _H)))


_stage1 = pl.pallas_call(
    _stage1_body,
    grid=(_GRID,),
    in_specs=[
        pl.BlockSpec((_RB, 128), lambda i: (i, 0)),
        pl.BlockSpec((128, _H), lambda i: (0, 0)),
        pl.BlockSpec((1, _H), lambda i: (0, 0)),
        pl.BlockSpec((_NC, _RB, 1), lambda i: (0, i, 0)),
    ],
    out_specs=[
        pl.BlockSpec((_RB, _H), lambda i: (i, 0)),
        pl.BlockSpec((_RB, _W), lambda i: (i, 0)),
    ],
    out_shape=[
        jax.ShapeDtypeStruct((_N, _H), jnp.float32),
        jax.ShapeDtypeStruct((_N, _W), jnp.float32),
    ],
)


def _mid_body(h_ref, parts_ref, degp_ref, wt_ref, wb_ref, aa_ref, ab_ref,
              hn_ref, gn_ref):
    h = h_ref[...]
    dinv = _dinv_from(degp_ref)
    c = dinv * (parts_ref[0, :, :_H] + parts_ref[1, :, :_H])
    z = (jnp.dot(h, wt_ref[...], preferred_element_type=jnp.float32)
         + jnp.dot(c, wb_ref[...], preferred_element_type=jnp.float32)
         + jnp.dot(h, aa_ref[...], preferred_element_type=jnp.float32)
         * jnp.dot(h, ab_ref[...], preferred_element_type=jnp.float32))
    hn = jnp.maximum(z, 0.0)
    hn_ref[...] = hn
    gn_ref[...] = jnp.pad(dinv * hn, ((0, 0), (0, _W - _H)))


def _last_body(h_ref, parts_ref, degp_ref, wt_ref, wb_ref, aa_ref, ab_ref,
               out_ref):
    h = h_ref[...]
    dinv = _dinv_from(degp_ref)
    c = dinv * (parts_ref[0, :, :_H] + parts_ref[1, :, :_H])
    z = (jnp.dot(h, wt_ref[...], preferred_element_type=jnp.float32)
         + jnp.dot(c, wb_ref[...], preferred_element_type=jnp.float32)
         + jnp.dot(h, aa_ref[...], preferred_element_type=jnp.float32)
         * jnp.dot(h, ab_ref[...], preferred_element_type=jnp.float32))
    out_ref[...] = jnp.maximum(z, 0.0)


_mid_in_specs = [
    pl.BlockSpec((_RB, _H), lambda i: (i, 0)),
    pl.BlockSpec((_NC, _RB, _W), lambda i: (0, i, 0)),
    pl.BlockSpec((_NC, _RB, 1), lambda i: (0, i, 0)),
    pl.BlockSpec((_H, _H), lambda i: (0, 0)),
    pl.BlockSpec((_H, _H), lambda i: (0, 0)),
    pl.BlockSpec((_H, _H), lambda i: (0, 0)),
    pl.BlockSpec((_H, _H), lambda i: (0, 0)),
]

_stage_mid = pl.pallas_call(
    _mid_body,
    grid=(_GRID,),
    in_specs=_mid_in_specs,
    out_specs=[
        pl.BlockSpec((_RB, _H), lambda i: (i, 0)),
        pl.BlockSpec((_RB, _W), lambda i: (i, 0)),
    ],
    out_shape=[
        jax.ShapeDtypeStruct((_N, _H), jnp.float32),
        jax.ShapeDtypeStruct((_N, _W), jnp.float32),
    ],
)

_stage_last = pl.pallas_call(
    _last_body,
    grid=(_GRID,),
    in_specs=_mid_in_specs,
    out_specs=pl.BlockSpec((_RB, _H), lambda i: (i, 0)),
    out_shape=jax.ShapeDtypeStruct((_N, _H), jnp.float32),
)


def kernel(x, edge_index, W1, b1, W2, A2a, A2b, W3, A3a, A3b, W4, A4a, A4b):
    src = edge_index[0].astype(jnp.int32)
    dst = edge_index[1].astype(jnp.int32)
    deg_kernel = _get_deg_kernel()
    conv_kernel = _get_conv_kernel()
    deg_parts = deg_kernel(dst).reshape(_NC, _NPAD, 1)
    h1, g1 = _stage1(x, W1, b1.reshape(1, _H), deg_parts)
    p1 = conv_kernel(g1, src, dst)
    h2, g2 = _stage_mid(h1, p1, deg_parts, W2[:_H], W2[_H:], A2a, A2b)
    p2 = conv_kernel(g2, src, dst)
    h3, g3 = _stage_mid(h2, p2, deg_parts, W3[:_H], W3[_H:], A3a, A3b)
    p3 = conv_kernel(g3, src, dst)
    out = _stage_last(h3, p3, deg_parts, W4[:_H], W4[_H:], A4a, A4b)
    return out


# speedup vs baseline: 4.1145x; 1.1270x over previous
import functools

import jax
import jax.numpy as jnp
from jax import lax
from jax.experimental import pallas as pl
from jax.experimental.pallas import tpu as pltpu
from jax.experimental.pallas import tpu_sc as plsc

_N = 10000
_E = 320000
_H = 64

_NC = 2
_NS = 16
_NW = _NC * _NS
_NPAD = 10240
_RPT = _NPAD // _NS
_ZC = 32
_W = 128
_EB = 80
_EW = _E // _NW
_NCH = _EW // _EB

def _deg_body(dst_hbm, out_hbm, acc_sh, dst_i, ones_v, zero_v, isem, ssem):
    cid = lax.axis_index("c")
    sid = lax.axis_index("s")
    w = cid * _NS + sid
    rbase = sid * _RPT
    ebase = w * _EW

    def idx_start(cexpr, s8):
        pltpu.async_copy(dst_hbm.at[pl.ds(ebase + cexpr * _EB, _EB)],
                         dst_i.at[s8], isem)

    def idx_wait():
        pltpu.make_async_copy(dst_hbm.at[pl.ds(ebase, _EB)], dst_i.at[0],
                              isem).wait()

    def scatter_start(s8):
        pltpu.async_copy(ones_v, acc_sh.at[dst_i.at[s8]], ssem, add=True)

    def scatter_wait():
        pltpu.make_async_copy(ones_v, acc_sh.at[pl.ds(0, _EB)], ssem).wait()

    def chunk_ops(cexpr, cm, s3=True, s6=True):
        idx_wait()
        scatter_start(cm % 8)
        if s3:
            scatter_wait()
        if s6:
            idx_start(cexpr + 6, (cm + 6) % 8)

    for j in range(_EB // 16):
        ones_v[pl.ds(j * 16, 16)] = jnp.ones((16,), jnp.float32)
    for j in range(_RPT // 16):
        zero_v[pl.ds(j * 16, 16)] = jnp.zeros((16,), jnp.float32)
    pltpu.sync_copy(zero_v, acc_sh.at[pl.ds(rbase, _RPT)])
    plsc.subcore_barrier()

    for c in range(6):
        idx_start(c, c)
    chunk_ops(0, 0, s3=False)
    chunk_ops(1, 1, s3=False)

    def sloop(i, car):
        cbase = 2 + 8 * i
        for j in range(8):
            chunk_ops(cbase + j, 2 + j)
        return car
    lax.fori_loop(0, 14, sloop, 0)

    for c in range(114, 119):
        chunk_ops(c, c)
    for c in range(119, 125):
        chunk_ops(c, c, s6=False)
    scatter_wait()
    scatter_wait()

    plsc.subcore_barrier()
    pltpu.sync_copy(acc_sh.at[pl.ds(rbase, _RPT)],
                    out_hbm.at[cid, pl.ds(rbase, _RPT)])


@functools.cache
def _get_deg_kernel():
    mesh = plsc.VectorSubcoreMesh(core_axis_name="c", subcore_axis_name="s")
    return pl.kernel(
        _deg_body,
        out_type=jax.ShapeDtypeStruct((_NC, _NPAD), jnp.float32),
        mesh=mesh,
        scratch_types=[
            pltpu.VMEM_SHARED((_NPAD,), jnp.float32),
            pltpu.VMEM((8, _EB), jnp.int32),
            pltpu.VMEM((_EB,), jnp.float32),
            pltpu.VMEM((_RPT,), jnp.float32),
            pltpu.SemaphoreType.DMA,
            pltpu.SemaphoreType.DMA,
        ],
    )


def _conv_body(g_hbm, src_hbm, dst_hbm, out_hbm, acc_sh, src_i, dst_i, rows,
               isem, gsem, ssem):
    cid = lax.axis_index("c")
    sid = lax.axis_index("s")
    w = cid * _NS + sid
    rbase = sid * _RPT
    ebase = w * _EW

    def idx_start(cexpr, s8):
        off = ebase + cexpr * _EB
        pltpu.async_copy(src_hbm.at[pl.ds(off, _EB)], src_i.at[s8], isem)
        pltpu.async_copy(dst_hbm.at[pl.ds(off, _EB)], dst_i.at[s8], isem)

    def idx_wait():
        pltpu.make_async_copy(src_hbm.at[pl.ds(ebase, _EB)], src_i.at[0],
                              isem).wait()
        pltpu.make_async_copy(dst_hbm.at[pl.ds(ebase, _EB)], dst_i.at[0],
                              isem).wait()

    def gather_start(s8, b4):
        pltpu.async_copy(g_hbm.at[src_i.at[s8]], rows.at[b4], gsem)

    def gather_wait():
        pltpu.make_async_copy(g_hbm.at[pl.ds(0, _EB)], rows.at[0], gsem).wait()

    def scatter_start(b4, s8):
        pltpu.async_copy(rows.at[b4], acc_sh.at[dst_i.at[s8]], ssem, add=True)

    def scatter_wait():
        pltpu.make_async_copy(rows.at[0], acc_sh.at[pl.ds(0, _EB)], ssem).wait()

    def chunk_ops(cexpr, cm, s3=True, s45=True, s6=True):
        gather_wait()
        scatter_start(cm % 4, cm % 8)
        if s3:
            scatter_wait()
        if s45:
            idx_wait()
            gather_start((cm + 2) % 8, (cm + 2) % 4)
        if s6:
            idx_start(cexpr + 6, (cm + 6) % 8)

    def zfill(r, car):
        for j in range(_W // 16):
            rows[0, r, pl.ds(j * 16, 16)] = jnp.zeros((16,), jnp.float32)
        return car
    lax.fori_loop(0, _EB, zfill, 0)

    def zloop(k, car):
        pltpu.sync_copy(rows.at[0], acc_sh.at[pl.ds(rbase + k * _EB, _EB)])
        return car
    lax.fori_loop(0, _RPT // _EB, zloop, 0)
    plsc.subcore_barrier()

    pltpu.sync_copy(src_hbm.at[pl.ds(ebase, _EB)], src_i.at[0])
    pltpu.sync_copy(dst_hbm.at[pl.ds(ebase, _EB)], dst_i.at[0])
    pltpu.sync_copy(src_hbm.at[pl.ds(ebase + _EB, _EB)], src_i.at[1])
    pltpu.sync_copy(dst_hbm.at[pl.ds(ebase + _EB, _EB)], dst_i.at[1])
    for c in (2, 3, 4, 5):
        idx_start(c, c)
    gather_start(0, 0)
    gather_start(1, 1)
    chunk_ops(0, 0, s3=False)
    chunk_ops(1, 1, s3=False)

    def sloop(i, car):
        cbase = 2 + 8 * i
        for j in range(8):
            chunk_ops(cbase + j, 2 + j)
        return car
    lax.fori_loop(0, 14, sloop, 0)

    for c in range(114, 119):
        chunk_ops(c, c)
    for c in range(119, 123):
        chunk_ops(c, c, s6=False)
    for c in (123, 124):
        chunk_ops(c, c, s45=False, s6=False)
    scatter_wait()
    scatter_wait()

    plsc.subcore_barrier()
    pltpu.sync_copy(acc_sh.at[pl.ds(rbase, _RPT)],
                    out_hbm.at[cid, pl.ds(rbase, _RPT)])


@functools.cache
def _get_conv_kernel():
    mesh = plsc.VectorSubcoreMesh(core_axis_name="c", subcore_axis_name="s")
    return pl.kernel(
        _conv_body,
        out_type=jax.ShapeDtypeStruct((_NC, _NPAD, _W), jnp.float32),
        mesh=mesh,
        scratch_types=[
            pltpu.VMEM_SHARED((_NPAD, _W), jnp.float32),
            pltpu.VMEM((8, _EB), jnp.int32),
            pltpu.VMEM((8, _EB), jnp.int32),
            pltpu.VMEM((4, _EB, _W), jnp.float32),
            pltpu.SemaphoreType.DMA,
            pltpu.SemaphoreType.DMA,
            pltpu.SemaphoreType.DMA,
        ],
    )

_RB = 1000
_GRID = _N // _RB


def _dinv_from(degp_ref):
    deg = degp_ref[0] + degp_ref[1]
    return jnp.where(deg > 0, lax.rsqrt(jnp.maximum(deg, 1.0)), 0.0)


def _stage1_body(x_ref, w1_ref, b1_ref, degp_ref, h_ref, g_ref):
    h = jnp.maximum(
        jnp.dot(x_ref[...], w1_ref[...], preferred_element_type=jnp.float32)
        + b1_ref[...], 0.0)
    dinv = _dinv_from(degp_ref)
    h_ref[...] = h
    g_ref[...] = jnp.pad(dinv * h, ((0, 0), (0, _W - _H)))


_stage1 = pl.pallas_call(
    _stage1_body,
    grid=(_GRID,),
    in_specs=[
        pl.BlockSpec((_RB, 128), lambda i: (i, 0)),
        pl.BlockSpec((128, _H), lambda i: (0, 0)),
        pl.BlockSpec((1, _H), lambda i: (0, 0)),
        pl.BlockSpec((_NC, _RB, 1), lambda i: (0, i, 0)),
    ],
    out_specs=[
        pl.BlockSpec((_RB, _H), lambda i: (i, 0)),
        pl.BlockSpec((_RB, _W), lambda i: (i, 0)),
    ],
    out_shape=[
        jax.ShapeDtypeStruct((_N, _H), jnp.float32),
        jax.ShapeDtypeStruct((_N, _W), jnp.float32),
    ],
)


def _mid_body(h_ref, parts_ref, degp_ref, wt_ref, wb_ref, aa_ref, ab_ref,
              hn_ref, gn_ref):
    h = h_ref[...]
    dinv = _dinv_from(degp_ref)
    c = dinv * (parts_ref[0, :, :_H] + parts_ref[1, :, :_H])
    z = (jnp.dot(h, wt_ref[...], preferred_element_type=jnp.float32)
         + jnp.dot(c, wb_ref[...], preferred_element_type=jnp.float32)
         + jnp.dot(h, aa_ref[...], preferred_element_type=jnp.float32)
         * jnp.dot(h, ab_ref[...], preferred_element_type=jnp.float32))
    hn = jnp.maximum(z, 0.0)
    hn_ref[...] = hn
    gn_ref[...] = jnp.pad(dinv * hn, ((0, 0), (0, _W - _H)))


def _last_body(h_ref, parts_ref, degp_ref, wt_ref, wb_ref, aa_ref, ab_ref,
               out_ref):
    h = h_ref[...]
    dinv = _dinv_from(degp_ref)
    c = dinv * (parts_ref[0, :, :_H] + parts_ref[1, :, :_H])
    z = (jnp.dot(h, wt_ref[...], preferred_element_type=jnp.float32)
         + jnp.dot(c, wb_ref[...], preferred_element_type=jnp.float32)
         + jnp.dot(h, aa_ref[...], preferred_element_type=jnp.float32)
         * jnp.dot(h, ab_ref[...], preferred_element_type=jnp.float32))
    out_ref[...] = jnp.maximum(z, 0.0)


_mid_in_specs = [
    pl.BlockSpec((_RB, _H), lambda i: (i, 0)),
    pl.BlockSpec((_NC, _RB, _W), lambda i: (0, i, 0)),
    pl.BlockSpec((_NC, _RB, 1), lambda i: (0, i, 0)),
    pl.BlockSpec((_H, _H), lambda i: (0, 0)),
    pl.BlockSpec((_H, _H), lambda i: (0, 0)),
    pl.BlockSpec((_H, _H), lambda i: (0, 0)),
    pl.BlockSpec((_H, _H), lambda i: (0, 0)),
]

_stage_mid = pl.pallas_call(
    _mid_body,
    grid=(_GRID,),
    in_specs=_mid_in_specs,
    out_specs=[
        pl.BlockSpec((_RB, _H), lambda i: (i, 0)),
        pl.BlockSpec((_RB, _W), lambda i: (i, 0)),
    ],
    out_shape=[
        jax.ShapeDtypeStruct((_N, _H), jnp.float32),
        jax.ShapeDtypeStruct((_N, _W), jnp.float32),
    ],
)

_stage_last = pl.pallas_call(
    _last_body,
    grid=(_GRID,),
    in_specs=_mid_in_specs,
    out_specs=pl.BlockSpec((_RB, _H), lambda i: (i, 0)),
    out_shape=jax.ShapeDtypeStruct((_N, _H), jnp.float32),
)


def kernel(x, edge_index, W1, b1, W2, A2a, A2b, W3, A3a, A3b, W4, A4a, A4b):
    src = edge_index[0].astype(jnp.int32)
    dst = edge_index[1].astype(jnp.int32)
    deg_kernel = _get_deg_kernel()
    conv_kernel = _get_conv_kernel()
    deg_parts = deg_kernel(dst).reshape(_NC, _NPAD, 1)
    h1, g1 = _stage1(x, W1, b1.reshape(1, _H), deg_parts)
    p1 = conv_kernel(g1, src, dst)
    h2, g2 = _stage_mid(h1, p1, deg_parts, W2[:_H], W2[_H:], A2a, A2b)
    p2 = conv_kernel(g2, src, dst)
    h3, g3 = _stage_mid(h2, p2, deg_parts, W3[:_H], W3[_H:], A3a, A3b)
    p3 = conv_kernel(g3, src, dst)
    out = _stage_last(h3, p3, deg_parts, W4[:_H], W4[_H:], A4a, A4b)
    return out


# speedup vs baseline: 4.5396x; 1.1033x over previous
import functools

import jax
import jax.numpy as jnp
from jax import lax
from jax.experimental import pallas as pl
from jax.experimental.pallas import tpu as pltpu
from jax.experimental.pallas import tpu_sc as plsc

_N = 10000
_E = 320000
_H = 64

_NC = 2
_NS = 16
_NW = _NC * _NS
_NPAD = 10240
_RPT = _NPAD // _NS
_ZC = 32
_W = 128
_EB = 80
_EW = _E // _NW
_NCH = _EW // _EB

def _deg_body(dst_hbm, out_hbm, acc_sh, dst_i, ones_v, zero_v, isem, ssem):
    cid = lax.axis_index("c")
    sid = lax.axis_index("s")
    w = cid * _NS + sid
    rbase = sid * _RPT
    ebase = w * _EW

    def idx_start(cexpr, s8):
        pltpu.async_copy(dst_hbm.at[pl.ds(ebase + cexpr * _EB, _EB)],
                         dst_i.at[s8], isem)

    def idx_wait():
        pltpu.make_async_copy(dst_hbm.at[pl.ds(ebase, _EB)], dst_i.at[0],
                              isem).wait()

    def scatter_start(s8):
        pltpu.async_copy(ones_v, acc_sh.at[dst_i.at[s8]], ssem, add=True)

    def scatter_wait():
        pltpu.make_async_copy(ones_v, acc_sh.at[pl.ds(0, _EB)], ssem).wait()

    def chunk_ops(cexpr, cm, s3=True, s6=True):
        idx_wait()
        scatter_start(cm % 8)
        if s3:
            scatter_wait()
        if s6:
            idx_start(cexpr + 6, (cm + 6) % 8)

    for j in range(_EB // 16):
        ones_v[pl.ds(j * 16, 16)] = jnp.ones((16,), jnp.float32)
    for j in range(_RPT // 16):
        zero_v[pl.ds(j * 16, 16)] = jnp.zeros((16,), jnp.float32)
    pltpu.sync_copy(zero_v, acc_sh.at[pl.ds(rbase, _RPT)])
    plsc.subcore_barrier()

    for c in range(6):
        idx_start(c, c)
    chunk_ops(0, 0, s3=False)
    chunk_ops(1, 1, s3=False)

    def sloop(i, car):
        cbase = 2 + 8 * i
        for j in range(8):
            chunk_ops(cbase + j, 2 + j)
        return car
    lax.fori_loop(0, 14, sloop, 0)

    for c in range(114, 119):
        chunk_ops(c, c)
    for c in range(119, 125):
        chunk_ops(c, c, s6=False)
    scatter_wait()
    scatter_wait()

    plsc.subcore_barrier()
    pltpu.sync_copy(acc_sh.at[pl.ds(rbase, _RPT)],
                    out_hbm.at[cid, pl.ds(rbase, _RPT)])


@functools.cache
def _get_deg_kernel():
    mesh = plsc.VectorSubcoreMesh(core_axis_name="c", subcore_axis_name="s")
    return pl.kernel(
        _deg_body,
        out_type=jax.ShapeDtypeStruct((_NC, _NPAD), jnp.float32),
        mesh=mesh,
        scratch_types=[
            pltpu.VMEM_SHARED((_NPAD,), jnp.float32),
            pltpu.VMEM((8, _EB), jnp.int32),
            pltpu.VMEM((_EB,), jnp.float32),
            pltpu.VMEM((_RPT,), jnp.float32),
            pltpu.SemaphoreType.DMA,
            pltpu.SemaphoreType.DMA,
        ],
    )


def _conv_body(g_hbm, src_hbm, dst_hbm, out_hbm, acc_sh, src_i, dst_i, rows,
               isem, gsem, ssem):
    cid = lax.axis_index("c")
    sid = lax.axis_index("s")
    w = cid * _NS + sid
    rbase = sid * _RPT
    ebase = w * _EW

    def idx_start(cexpr, s8):
        off = ebase + cexpr * _EB
        pltpu.async_copy(src_hbm.at[pl.ds(off, _EB)], src_i.at[s8], isem)
        pltpu.async_copy(dst_hbm.at[pl.ds(off, _EB)], dst_i.at[s8], isem)

    def idx_wait():
        pltpu.make_async_copy(src_hbm.at[pl.ds(ebase, _EB)], src_i.at[0],
                              isem).wait()
        pltpu.make_async_copy(dst_hbm.at[pl.ds(ebase, _EB)], dst_i.at[0],
                              isem).wait()

    def gather_start(s8, b4):
        pltpu.async_copy(g_hbm.at[src_i.at[s8]], rows.at[b4], gsem)

    def gather_wait():
        pltpu.make_async_copy(g_hbm.at[pl.ds(0, _EB)], rows.at[0], gsem).wait()

    def scatter_start(b4, s8):
        pltpu.async_copy(rows.at[b4], acc_sh.at[dst_i.at[s8]], ssem, add=True)

    def scatter_wait():
        pltpu.make_async_copy(rows.at[0], acc_sh.at[pl.ds(0, _EB)], ssem).wait()

    def chunk_ops(cexpr, cm, s3=True, s45=True, s6=True):
        gather_wait()
        scatter_start(cm % 4, cm % 8)
        if s3:
            scatter_wait()
        if s45:
            idx_wait()
            gather_start((cm + 3) % 8, (cm + 3) % 4)
        if s6:
            idx_start(cexpr + 7, (cm + 7) % 8)

    for c in range(7):
        idx_start(c, c)

    def zfill(r, car):
        for j in range(_W // 16):
            rows[0, r, pl.ds(j * 16, 16)] = jnp.zeros((16,), jnp.float32)
        return car
    lax.fori_loop(0, _EB, zfill, 0)

    def zloop(k, car):
        pltpu.sync_copy(rows.at[0], acc_sh.at[pl.ds(rbase + k * _EB, _EB)])
        return car
    lax.fori_loop(0, _RPT // _EB, zloop, 0)
    plsc.subcore_barrier()

    idx_wait()
    gather_start(0, 0)
    idx_wait()
    gather_start(1, 1)
    idx_wait()
    gather_start(2, 2)
    gather_wait()
    scatter_start(0, 0)
    idx_wait()
    gather_start(3, 3)
    idx_start(7, 7)
    gather_wait()
    scatter_start(1, 1)
    scatter_wait()
    idx_wait()
    gather_start(4 % 8, 4 % 4)
    idx_start(8, 0)

    def sloop(i, car):
        cbase = 2 + 8 * i
        for j in range(8):
            chunk_ops(cbase + j, 2 + j)
        return car
    lax.fori_loop(0, 14, sloop, 0)

    for c in range(114, 118):
        chunk_ops(c, c)
    for c in range(118, 122):
        chunk_ops(c, c, s6=False)
    for c in (122, 123, 124):
        chunk_ops(c, c, s45=False, s6=False)
    scatter_wait()

    plsc.subcore_barrier()
    pltpu.sync_copy(acc_sh.at[pl.ds(rbase, _RPT)],
                    out_hbm.at[cid, pl.ds(rbase, _RPT)])


@functools.cache
def _get_conv_kernel():
    mesh = plsc.VectorSubcoreMesh(core_axis_name="c", subcore_axis_name="s")
    return pl.kernel(
        _conv_body,
        out_type=jax.ShapeDtypeStruct((_NC, _NPAD, _W), jnp.float32),
        mesh=mesh,
        scratch_types=[
            pltpu.VMEM_SHARED((_NPAD, _W), jnp.float32),
            pltpu.VMEM((8, _EB), jnp.int32),
            pltpu.VMEM((8, _EB), jnp.int32),
            pltpu.VMEM((4, _EB, _W), jnp.float32),
            pltpu.SemaphoreType.DMA,
            pltpu.SemaphoreType.DMA,
            pltpu.SemaphoreType.DMA,
        ],
    )

_RB = 1000
_GRID = _N // _RB


def _dinv_from(degp_ref):
    deg = degp_ref[0] + degp_ref[1]
    return jnp.where(deg > 0, lax.rsqrt(jnp.maximum(deg, 1.0)), 0.0)


def _stage1_body(x_ref, w1_ref, b1_ref, degp_ref, h_ref, g_ref):
    h = jnp.maximum(
        jnp.dot(x_ref[...], w1_ref[...], preferred_element_type=jnp.float32)
        + b1_ref[...], 0.0)
    dinv = _dinv_from(degp_ref)
    h_ref[...] = h
    g_ref[...] = jnp.pad(dinv * h, ((0, 0), (0, _W - _H)))


_stage1 = pl.pallas_call(
    _stage1_body,
    grid=(_GRID,),
    in_specs=[
        pl.BlockSpec((_RB, 128), lambda i: (i, 0)),
        pl.BlockSpec((128, _H), lambda i: (0, 0)),
        pl.BlockSpec((1, _H), lambda i: (0, 0)),
        pl.BlockSpec((_NC, _RB, 1), lambda i: (0, i, 0)),
    ],
    out_specs=[
        pl.BlockSpec((_RB, _H), lambda i: (i, 0)),
        pl.BlockSpec((_RB, _W), lambda i: (i, 0)),
    ],
    out_shape=[
        jax.ShapeDtypeStruct((_N, _H), jnp.float32),
        jax.ShapeDtypeStruct((_N, _W), jnp.float32),
    ],
)


def _mid_body(h_ref, parts_ref, degp_ref, wt_ref, wb_ref, aa_ref, ab_ref,
              hn_ref, gn_ref):
    h = h_ref[...]
    dinv = _dinv_from(degp_ref)
    c = dinv * (parts_ref[0, :, :_H] + parts_ref[1, :, :_H])
    z = (jnp.dot(h, wt_ref[...], preferred_element_type=jnp.float32)
         + jnp.dot(c, wb_ref[...], preferred_element_type=jnp.float32)
         + jnp.dot(h, aa_ref[...], preferred_element_type=jnp.float32)
         * jnp.dot(h, ab_ref[...], preferred_element_type=jnp.float32))
    hn = jnp.maximum(z, 0.0)
    hn_ref[...] = hn
    gn_ref[...] = jnp.pad(dinv * hn, ((0, 0), (0, _W - _H)))


def _last_body(h_ref, parts_ref, degp_ref, wt_ref, wb_ref, aa_ref, ab_ref,
               out_ref):
    h = h_ref[...]
    dinv = _dinv_from(degp_ref)
    c = dinv * (parts_ref[0, :, :_H] + parts_ref[1, :, :_H])
    z = (jnp.dot(h, wt_ref[...], preferred_element_type=jnp.float32)
         + jnp.dot(c, wb_ref[...], preferred_element_type=jnp.float32)
         + jnp.dot(h, aa_ref[...], preferred_element_type=jnp.float32)
         * jnp.dot(h, ab_ref[...], preferred_element_type=jnp.float32))
    out_ref[...] = jnp.maximum(z, 0.0)


_mid_in_specs = [
    pl.BlockSpec((_RB, _H), lambda i: (i, 0)),
    pl.BlockSpec((_NC, _RB, _W), lambda i: (0, i, 0)),
    pl.BlockSpec((_NC, _RB, 1), lambda i: (0, i, 0)),
    pl.BlockSpec((_H, _H), lambda i: (0, 0)),
    pl.BlockSpec((_H, _H), lambda i: (0, 0)),
    pl.BlockSpec((_H, _H), lambda i: (0, 0)),
    pl.BlockSpec((_H, _H), lambda i: (0, 0)),
]

_stage_mid = pl.pallas_call(
    _mid_body,
    grid=(_GRID,),
    in_specs=_mid_in_specs,
    out_specs=[
        pl.BlockSpec((_RB, _H), lambda i: (i, 0)),
        pl.BlockSpec((_RB, _W), lambda i: (i, 0)),
    ],
    out_shape=[
        jax.ShapeDtypeStruct((_N, _H), jnp.float32),
        jax.ShapeDtypeStruct((_N, _W), jnp.float32),
    ],
)

_stage_last = pl.pallas_call(
    _last_body,
    grid=(_GRID,),
    in_specs=_mid_in_specs,
    out_specs=pl.BlockSpec((_RB, _H), lambda i: (i, 0)),
    out_shape=jax.ShapeDtypeStruct((_N, _H), jnp.float32),
)


def kernel(x, edge_index, W1, b1, W2, A2a, A2b, W3, A3a, A3b, W4, A4a, A4b):
    src = edge_index[0].astype(jnp.int32)
    dst = edge_index[1].astype(jnp.int32)
    deg_kernel = _get_deg_kernel()
    conv_kernel = _get_conv_kernel()
    deg_parts = deg_kernel(dst).reshape(_NC, _NPAD, 1)
    h1, g1 = _stage1(x, W1, b1.reshape(1, _H), deg_parts)
    p1 = conv_kernel(g1, src, dst)
    h2, g2 = _stage_mid(h1, p1, deg_parts, W2[:_H], W2[_H:], A2a, A2b)
    p2 = conv_kernel(g2, src, dst)
    h3, g3 = _stage_mid(h2, p2, deg_parts, W3[:_H], W3[_H:], A3a, A3b)
    p3 = conv_kernel(g3, src, dst)
    out = _stage_last(h3, p3, deg_parts, W4[:_H], W4[_H:], A4a, A4b)
    return out
